# baseline (speedup 1.0000x reference)
import functools

import jax
import jax.numpy as jnp
from jax import lax
from jax.experimental import pallas as pl
from jax.experimental.pallas import tpu as pltpu
from jax.experimental.pallas import tpu_sc as plsc

N_NODES = 10000
N_EDGES = 160000
HEADS = 6
HID = 512
HH = HEADS * HID
NPAD = 10240

NW = 32
E_TOT = N_EDGES + N_NODES
EPT = ((E_TOT + NW * 32 - 1) // (NW * 32)) * 32
E_PAD = EPT * NW
GROUPS = EPT // 16
CCH = HID // 16

_SELU_SCALE = 1.0507009873554805
_SELU_ALPHA = 1.6732632423543772


def _selu(y):
    return _SELU_SCALE * jnp.where(y > 0, y, _SELU_ALPHA * (jnp.exp(y) - 1.0))


def _mm_body(x_ref, w_ref, b_ref, o_ref, *, act):
    y = jnp.dot(x_ref[...], w_ref[...], preferred_element_type=jnp.float32)
    y = y + b_ref[...]
    if act:
        y = _selu(y)
    o_ref[...] = y


def _matmul(x, w, b, act=False, bm=None):
    m, k = x.shape
    n = w.shape[1]
    if bm is None or bm >= m:
        bm = m
    assert m % bm == 0
    return pl.pallas_call(
        functools.partial(_mm_body, act=act),
        grid=(m // bm,),
        in_specs=[
            pl.BlockSpec((bm, k), lambda i: (i, 0)),
            pl.BlockSpec((k, n), lambda i: (0, 0)),
            pl.BlockSpec((1, n), lambda i: (0, 0)),
        ],
        out_specs=pl.BlockSpec((bm, n), lambda i: (i, 0)),
        out_shape=jax.ShapeDtypeStruct((m, n), jnp.float32),
    )(x, w, b.reshape(1, n))


def _conv3x3(img, w, b):
    c = img.shape[0]
    o = w.shape[0]
    xp = jnp.pad(img, ((0, 0), (1, 1), (1, 1)))
    cols = [xp[:, dy:dy + 64, dx:dx + 64] for dy in range(3) for dx in range(3)]
    pat = jnp.stack(cols, axis=1).reshape(c * 9, 4096).T
    wm = w.reshape(o, c * 9).T
    y = _matmul(pat, wm, b, act=True)
    return y


def _tail_body(g_ref, bias_ref, cw_ref, cb_ref, coord_ref, o_ref):
    h = _selu(g_ref[...] / HEADS + bias_ref[...])
    oc = jnp.dot(h, cw_ref[...], preferred_element_type=jnp.float32)
    oc = oc + cb_ref[...]
    coord = coord_ref[...]
    upper = coord[:, 0:1] == 1.0
    down = coord[:, 0:1] == 0.0
    left = coord[:, 1:2] == 0.0
    right = coord[:, 1:2] == 1.0
    ox = jnp.where(upper, 1.0, oc[:, 0:1])
    ox = jnp.where(down, 0.0, ox)
    oy = jnp.where(left, 0.0, oc[:, 1:2])
    oy = jnp.where(right, 1.0, oy)
    o_ref[...] = jnp.concatenate([ox, oy], axis=1)


def _tail(gat, a_bias, c_w, c_b, coord):
    bm = 1000
    return pl.pallas_call(
        _tail_body,
        grid=(N_NODES // bm,),
        in_specs=[
            pl.BlockSpec((bm, HID), lambda i: (i, 0)),
            pl.BlockSpec((1, HID), lambda i: (0, 0)),
            pl.BlockSpec((HID, 2), lambda i: (0, 0)),
            pl.BlockSpec((1, 2), lambda i: (0, 0)),
            pl.BlockSpec((bm, 2), lambda i: (i, 0)),
        ],
        out_specs=pl.BlockSpec((bm, 2), lambda i: (i, 0)),
        out_shape=jax.ShapeDtypeStruct((N_NODES, 2), jnp.float32),
    )(gat, a_bias.reshape(1, HID), c_w, c_b.reshape(1, 2), coord)


def _gat_sc_body(xl_hbm, xr_hbm, att_hbm, src_hbm, dst_hbm,
                 out_hbm, pacc_hbm, pden_hbm,
                 idx_v, dstv_v, u_buf, v_buf, att_v, acc_v, den_v, orow_v,
                 state_s, sem, sem2):
    wid = lax.axis_index("s") * 2 + lax.axis_index("c")
    base = wid * EPT
    zero16 = jnp.zeros((16,), jnp.float32)

    pltpu.sync_copy(src_hbm.at[pl.ds(base, EPT)], idx_v)
    pltpu.sync_copy(dst_hbm.at[pl.ds(base, EPT)], dstv_v.at[pl.ds(0, EPT)])
    pltpu.sync_copy(att_hbm, att_v)

    def _zero_acc():
        def zb(i, _):
            acc_v[pl.ds(i * 16, 16)] = zero16
            return 0
        lax.fori_loop(0, HH // 16, zb, 0)
        for h in range(HEADS):
            den_v[pl.ds(h * 16, 16)] = zero16

    _zero_acc()
    d0 = dstv_v[pl.ds(0, 16)][0]
    state_s[0] = d0
    state_s[1] = 0
    pltpu.sync_copy(xr_hbm.at[d0], v_buf)

    def _write_partial(cur, slot):
        pltpu.sync_copy(acc_v, pacc_hbm.at[slot])
        den_v[pl.ds(96, 16)] = zero16 + cur.astype(jnp.float32)
        den_v[pl.ds(112, 16)] = zero16 + 1.0
        pltpu.sync_copy(den_v, pden_hbm.at[slot])

    def _write_final(cur):
        def cb(c, _):
            s = zero16
            for h in range(HEADS):
                s = s + acc_v[pl.ds(h * HID + c * 16, 16)] / den_v[pl.ds(h * 16, 16)]
            orow_v[pl.ds(c * 16, 16)] = s
            return 0
        lax.fori_loop(0, CCH, cb, 0)
        pltpu.sync_copy(orow_v, out_hbm.at[cur])

    def _make_edge(slot):
        def _edge(j, g):
            eidx = pl.multiple_of(g * 16, 16) + j
            d = dstv_v[pl.ds(eidx, 16)][0]
            cur = state_s[0]

            @pl.when(d != cur)
            def _change():
                first = state_s[1] == 0

                def br_first(_):
                    _write_partial(cur, 2 * wid)
                    return 0

                def br_final(_):
                    _write_final(cur)
                    return 0

                lax.cond(first, br_first, br_final, 0)
                state_s[1] = 1
                _zero_acc()
                pltpu.sync_copy(xr_hbm.at[d], v_buf)
                state_s[0] = d

            for h in range(HEADS):
                hoff = h * HID
                ps = [zero16, zero16, zero16, zero16]
                for c in range(CCH):
                    off = hoff + c * 16
                    uu = u_buf[slot, j, pl.ds(off, 16)]
                    vv = v_buf[pl.ds(off, 16)]
                    aa = att_v[pl.ds(off, 16)]
                    s = uu + vv
                    s = jnp.maximum(s, 0.2 * s)
                    ps[c % 4] = ps[c % 4] + aa * s
                logit = jnp.sum((ps[0] + ps[1]) + (ps[2] + ps[3]))
                w = jnp.exp(zero16 + logit)
                den_v[pl.ds(h * 16, 16)] = den_v[pl.ds(h * 16, 16)] + w
                for c in range(CCH):
                    off = hoff + c * 16
                    acc_v[pl.ds(off, 16)] = acc_v[pl.ds(off, 16)] + w * u_buf[slot, j, pl.ds(off, 16)]
            return g
        return _edge

    _edge0 = _make_edge(0)
    _edge1 = _make_edge(1)

    def _start(g, slot, s):
        ivec = idx_v[pl.ds(pl.multiple_of(g * 16, 16), 16)]
        pltpu.async_copy(xl_hbm.at[ivec], u_buf.at[slot], s)

    def _drain(slot, s):
        pltpu.make_async_copy(xl_hbm.at[pl.ds(0, 16)], u_buf.at[slot], s).wait()

    _start(0, 0, sem)

    def _gpair(i, _):
        g0 = i * 2
        _start(g0 + 1, 1, sem2)
        _drain(0, sem)
        lax.fori_loop(0, 16, _edge0, g0)

        @pl.when(g0 + 2 < GROUPS)
        def _():
            _start(g0 + 2, 0, sem)

        _drain(1, sem2)
        lax.fori_loop(0, 16, _edge1, g0 + 1)
        return 0

    lax.fori_loop(0, GROUPS // 2, _gpair, 0)

    _write_partial(state_s[0], 2 * wid + 1)

    @pl.when(state_s[1] == 0)
    def _no_flush():
        def zb(i, _):
            acc_v[pl.ds(i * 16, 16)] = zero16
            return 0
        lax.fori_loop(0, HH // 16, zb, 0)
        pltpu.sync_copy(acc_v, pacc_hbm.at[2 * wid])
        den_v[pl.ds(96, 16)] = zero16
        den_v[pl.ds(112, 16)] = zero16
        pltpu.sync_copy(den_v, pden_hbm.at[2 * wid])


def _gat_sparsecore(xlp, xrp, att_flat, srcs_p, dsts_p):
    mesh = plsc.VectorSubcoreMesh(core_axis_name="c", subcore_axis_name="s")
    f32 = jnp.float32
    kern = pl.kernel(
        _gat_sc_body,
        out_type=[
            jax.ShapeDtypeStruct((N_NODES, HID), f32),
            jax.ShapeDtypeStruct((2 * NW, HH), f32),
            jax.ShapeDtypeStruct((2 * NW, 128), f32),
        ],
        mesh=mesh,
        compiler_params=pltpu.CompilerParams(needs_layout_passes=False),
        scratch_types=[
            pltpu.VMEM((EPT,), jnp.int32),
            pltpu.VMEM((EPT + 16,), jnp.int32),
            pltpu.VMEM((2, 16, HH), f32),
            pltpu.VMEM((HH,), f32),
            pltpu.VMEM((HH,), f32),
            pltpu.VMEM((HH,), f32),
            pltpu.VMEM((128,), f32),
            pltpu.VMEM((HID,), f32),
            pltpu.SMEM((8,), jnp.int32),
            pltpu.SemaphoreType.DMA,
            pltpu.SemaphoreType.DMA,
        ],
    )
    return kern(xlp, xrp, att_flat, srcs_p, dsts_p)


def _combine_partials(gat, pacc, pden):
    den = pden[:, 0:HEADS * 16:16]
    pdst = pden[:, 96].astype(jnp.int32)
    valid = pden[:, 112] > 0.5
    okay = valid & (pdst >= 0) & (pdst < N_NODES)
    eq = (pdst[:, None] == pdst[None, :]) & okay[:, None] & okay[None, :]
    eqf = eq.astype(jnp.float32)
    acc_c = eqf @ pacc
    den_c = eqf @ den
    den_c = jnp.where(den_c == 0.0, 1.0, den_c)
    rows = (acc_c.reshape(-1, HEADS, HID) / den_c[:, :, None]).sum(axis=1)
    safe = jnp.where(okay, pdst, N_NODES)
    return gat.at[safe].set(rows, mode="drop")


def kernel(x, conv_feat, mesh_feat, edge_index, node_num, bd_mask, poly_mesh,
           g_w1, g_b1, g_w2, g_b2, g_w3, g_b3, g_w4, g_b4,
           l_w1, l_b1, l_w2, l_b2, l_w3, l_b3, lin_w, lin_b,
           a_wl, a_bl, a_wr, a_br, a_att, a_bias, c_w, c_b):
    coord = x[:, :2]

    h = _conv3x3(conv_feat[0], g_w1, g_b1)
    h = _conv3x3(h.T.reshape(32, 64, 64), g_w2, g_b2)
    h = _conv3x3(h.T.reshape(64, 64, 64), g_w3, g_b3)
    h = _conv3x3(h.T.reshape(128, 64, 64), g_w4, g_b4)
    gfeat_vec = jnp.mean(h, axis=0)
    gfeat = jnp.broadcast_to(gfeat_vec[None, :], (N_NODES, 16))

    src0 = edge_index[0]
    dst0 = edge_index[1]
    m = jnp.concatenate([mesh_feat[dst0], mesh_feat[src0]], axis=1)
    eh = _matmul(m, l_w1, l_b1, act=True, bm=4000)
    eh = _matmul(eh, l_w2, l_b2, act=True, bm=4000)
    eh = _matmul(eh, l_w3, l_b3, act=True, bm=4000)
    lfeat = jax.ops.segment_sum(eh, dst0, num_segments=N_NODES)

    hidden_in = jnp.concatenate([x[:, 2:], lfeat, gfeat], axis=1)
    hidden = _matmul(hidden_in, lin_w, lin_b, act=True, bm=1000)

    xin = jnp.concatenate([coord, hidden], axis=1)
    xin_p = jnp.pad(xin, ((0, NPAD - N_NODES), (0, 0)))
    xlp = _matmul(xin_p, a_wl, a_bl, bm=1024)
    xrp = _matmul(xin_p, a_wr, a_br, bm=1024)

    loops = jnp.arange(N_NODES, dtype=edge_index.dtype)
    e_src = jnp.concatenate([src0, loops])
    e_dst = jnp.concatenate([dst0, loops])
    order = jnp.argsort(e_dst)
    srcs_s = e_src[order].astype(jnp.int32)
    dsts_s = e_dst[order].astype(jnp.int32)
    srcs_p = jnp.concatenate(
        [srcs_s, jnp.zeros((E_PAD - E_TOT,), jnp.int32)])
    dsts_p = jnp.concatenate(
        [dsts_s, jnp.full((E_PAD - E_TOT,), N_NODES, jnp.int32)])

    gat, pacc, pden = _gat_sparsecore(
        xlp, xrp, a_att.reshape(HH), srcs_p, dsts_p)
    gat = _combine_partials(gat, pacc, pden)

    return _tail(gat, a_bias, c_w, c_b, coord)


# speedup vs baseline: 1.1581x; 1.1581x over previous
import functools

import jax
import jax.numpy as jnp
from jax import lax
from jax.experimental import pallas as pl
from jax.experimental.pallas import tpu as pltpu
from jax.experimental.pallas import tpu_sc as plsc

N_NODES = 10000
N_EDGES = 160000
HEADS = 6
HID = 512
HH = HEADS * HID
NPAD = 10240

NW = 32
E_TOT = N_EDGES + N_NODES
EPT = ((E_TOT + NW * 32 - 1) // (NW * 32)) * 32
E_PAD = EPT * NW
GROUPS = EPT // 16
CCH = HID // 16

_SELU_SCALE = 1.0507009873554805
_SELU_ALPHA = 1.6732632423543772


def _selu(y):
    return _SELU_SCALE * jnp.where(y > 0, y, _SELU_ALPHA * (jnp.exp(y) - 1.0))


def _mm_body(x_ref, w_ref, b_ref, o_ref, *, act):
    y = jnp.dot(x_ref[...], w_ref[...], preferred_element_type=jnp.float32)
    y = y + b_ref[...]
    if act:
        y = _selu(y)
    o_ref[...] = y


def _matmul(x, w, b, act=False, bm=None):
    m, k = x.shape
    n = w.shape[1]
    if bm is None or bm >= m:
        bm = m
    assert m % bm == 0
    return pl.pallas_call(
        functools.partial(_mm_body, act=act),
        grid=(m // bm,),
        in_specs=[
            pl.BlockSpec((bm, k), lambda i: (i, 0)),
            pl.BlockSpec((k, n), lambda i: (0, 0)),
            pl.BlockSpec((1, n), lambda i: (0, 0)),
        ],
        out_specs=pl.BlockSpec((bm, n), lambda i: (i, 0)),
        out_shape=jax.ShapeDtypeStruct((m, n), jnp.float32),
    )(x, w, b.reshape(1, n))


def _conv3x3(img, w, b):
    c = img.shape[0]
    o = w.shape[0]
    xp = jnp.pad(img, ((0, 0), (1, 1), (1, 1)))
    cols = [xp[:, dy:dy + 64, dx:dx + 64] for dy in range(3) for dx in range(3)]
    pat = jnp.stack(cols, axis=1).reshape(c * 9, 4096).T
    wm = w.reshape(o, c * 9).T
    y = _matmul(pat, wm, b, act=True)
    return y


def _tail_body(g_ref, bias_ref, cw_ref, cb_ref, coord_ref, o_ref):
    h = _selu(g_ref[...] / HEADS + bias_ref[...])
    oc = jnp.dot(h, cw_ref[...], preferred_element_type=jnp.float32)
    oc = oc + cb_ref[...]
    coord = coord_ref[...]
    upper = coord[:, 0:1] == 1.0
    down = coord[:, 0:1] == 0.0
    left = coord[:, 1:2] == 0.0
    right = coord[:, 1:2] == 1.0
    ox = jnp.where(upper, 1.0, oc[:, 0:1])
    ox = jnp.where(down, 0.0, ox)
    oy = jnp.where(left, 0.0, oc[:, 1:2])
    oy = jnp.where(right, 1.0, oy)
    o_ref[...] = jnp.concatenate([ox, oy], axis=1)


def _tail(gat, a_bias, c_w, c_b, coord):
    bm = 1000
    return pl.pallas_call(
        _tail_body,
        grid=(N_NODES // bm,),
        in_specs=[
            pl.BlockSpec((bm, HID), lambda i: (i, 0)),
            pl.BlockSpec((1, HID), lambda i: (0, 0)),
            pl.BlockSpec((HID, 2), lambda i: (0, 0)),
            pl.BlockSpec((1, 2), lambda i: (0, 0)),
            pl.BlockSpec((bm, 2), lambda i: (i, 0)),
        ],
        out_specs=pl.BlockSpec((bm, 2), lambda i: (i, 0)),
        out_shape=jax.ShapeDtypeStruct((N_NODES, 2), jnp.float32),
    )(gat, a_bias.reshape(1, HID), c_w, c_b.reshape(1, 2), coord)


def _gat_sc_body(xl_hbm, xr_hbm, att_hbm, src_hbm, dst_hbm,
                 out_hbm, pacc_hbm, pden_hbm,
                 idx_v, dstv_v, u_buf, v_buf, att_v, acc_v, den_v, orow_v,
                 state_s, sem, sem2):
    wid = lax.axis_index("s") * 2 + lax.axis_index("c")
    base = wid * EPT
    zero16 = jnp.zeros((16,), jnp.float32)

    pltpu.sync_copy(src_hbm.at[pl.ds(base, EPT)], idx_v)
    pltpu.sync_copy(dst_hbm.at[pl.ds(base, EPT)], dstv_v.at[pl.ds(0, EPT)])
    pltpu.sync_copy(att_hbm, att_v)

    def _zero_acc():
        def zb(i, _):
            acc_v[pl.ds(i * 16, 16)] = zero16
            return 0
        lax.fori_loop(0, HH // 16, zb, 0)
        for h in range(HEADS):
            den_v[pl.ds(h * 16, 16)] = zero16

    _zero_acc()
    d0 = dstv_v[pl.ds(0, 16)][0]
    state_s[0] = d0
    state_s[1] = 0
    pltpu.sync_copy(xr_hbm.at[d0], v_buf)

    def _write_partial(cur, slot):
        pltpu.sync_copy(acc_v, pacc_hbm.at[slot])
        den_v[pl.ds(96, 16)] = zero16 + cur.astype(jnp.float32)
        den_v[pl.ds(112, 16)] = zero16 + 1.0
        pltpu.sync_copy(den_v, pden_hbm.at[slot])

    def _write_final(cur):
        def cb(c, _):
            s = zero16
            for h in range(HEADS):
                s = s + acc_v[pl.ds(h * HID + c * 16, 16)] / den_v[pl.ds(h * 16, 16)]
            orow_v[pl.ds(c * 16, 16)] = s
            return 0
        lax.fori_loop(0, CCH, cb, 0)
        pltpu.sync_copy(orow_v, out_hbm.at[cur])

    def _make_edge(slot):
        def _edge(j, g):
            eidx = pl.multiple_of(g * 16, 16) + j
            d = dstv_v[pl.ds(eidx, 16)][0]
            cur = state_s[0]

            @pl.when(d != cur)
            def _change():
                first = state_s[1] == 0

                def br_first(_):
                    _write_partial(cur, 2 * wid)
                    return 0

                def br_final(_):
                    _write_final(cur)
                    return 0

                lax.cond(first, br_first, br_final, 0)
                state_s[1] = 1
                _zero_acc()
                pltpu.sync_copy(xr_hbm.at[d], v_buf)
                state_s[0] = d

            for h in range(HEADS):
                hoff = h * HID
                ps = [zero16, zero16, zero16, zero16]
                for c in range(CCH):
                    off = hoff + c * 16
                    uu = u_buf[slot, j, pl.ds(off, 16)]
                    vv = v_buf[pl.ds(off, 16)]
                    aa = att_v[pl.ds(off, 16)]
                    s = uu + vv
                    s = jnp.maximum(s, 0.2 * s)
                    ps[c % 4] = ps[c % 4] + aa * s
                logit = jnp.sum((ps[0] + ps[1]) + (ps[2] + ps[3]))
                w = jnp.exp(zero16 + logit)
                den_v[pl.ds(h * 16, 16)] = den_v[pl.ds(h * 16, 16)] + w
                for c in range(CCH):
                    off = hoff + c * 16
                    acc_v[pl.ds(off, 16)] = acc_v[pl.ds(off, 16)] + w * u_buf[slot, j, pl.ds(off, 16)]
            return g
        return _edge

    _edge0 = _make_edge(0)

    def _group(g, _):
        ivec = idx_v[pl.ds(pl.multiple_of(g * 16, 16), 16)]
        pltpu.async_copy(xl_hbm.at[ivec], u_buf.at[0], sem).wait()
        lax.fori_loop(0, 16, _edge0, g)
        return 0

    lax.fori_loop(0, GROUPS, _group, 0)

    _write_partial(state_s[0], 2 * wid + 1)

    @pl.when(state_s[1] == 0)
    def _no_flush():
        def zb(i, _):
            acc_v[pl.ds(i * 16, 16)] = zero16
            return 0
        lax.fori_loop(0, HH // 16, zb, 0)
        pltpu.sync_copy(acc_v, pacc_hbm.at[2 * wid])
        den_v[pl.ds(96, 16)] = zero16
        den_v[pl.ds(112, 16)] = zero16
        pltpu.sync_copy(den_v, pden_hbm.at[2 * wid])


def _gat_sparsecore(xlp, xrp, att_flat, srcs_p, dsts_p):
    mesh = plsc.VectorSubcoreMesh(core_axis_name="c", subcore_axis_name="s")
    f32 = jnp.float32
    kern = pl.kernel(
        _gat_sc_body,
        out_type=[
            jax.ShapeDtypeStruct((N_NODES, HID), f32),
            jax.ShapeDtypeStruct((2 * NW, HH), f32),
            jax.ShapeDtypeStruct((2 * NW, 128), f32),
        ],
        mesh=mesh,
        compiler_params=pltpu.CompilerParams(needs_layout_passes=False),
        scratch_types=[
            pltpu.VMEM((EPT,), jnp.int32),
            pltpu.VMEM((EPT + 16,), jnp.int32),
            pltpu.VMEM((2, 16, HH), f32),
            pltpu.VMEM((HH,), f32),
            pltpu.VMEM((HH,), f32),
            pltpu.VMEM((HH,), f32),
            pltpu.VMEM((128,), f32),
            pltpu.VMEM((HID,), f32),
            pltpu.SMEM((8,), jnp.int32),
            pltpu.SemaphoreType.DMA,
            pltpu.SemaphoreType.DMA,
        ],
    )
    return kern(xlp, xrp, att_flat, srcs_p, dsts_p)


def _combine_partials(gat, pacc, pden):
    den = pden[:, 0:HEADS * 16:16]
    pdst = pden[:, 96].astype(jnp.int32)
    valid = pden[:, 112] > 0.5
    okay = valid & (pdst >= 0) & (pdst < N_NODES)
    eq = (pdst[:, None] == pdst[None, :]) & okay[:, None] & okay[None, :]
    eqf = eq.astype(jnp.float32)
    acc_c = eqf @ pacc
    den_c = eqf @ den
    den_c = jnp.where(den_c == 0.0, 1.0, den_c)
    rows = (acc_c.reshape(-1, HEADS, HID) / den_c[:, :, None]).sum(axis=1)
    safe = jnp.where(okay, pdst, N_NODES)
    return gat.at[safe].set(rows, mode="drop")


def kernel(x, conv_feat, mesh_feat, edge_index, node_num, bd_mask, poly_mesh,
           g_w1, g_b1, g_w2, g_b2, g_w3, g_b3, g_w4, g_b4,
           l_w1, l_b1, l_w2, l_b2, l_w3, l_b3, lin_w, lin_b,
           a_wl, a_bl, a_wr, a_br, a_att, a_bias, c_w, c_b):
    coord = x[:, :2]

    h = _conv3x3(conv_feat[0], g_w1, g_b1)
    h = _conv3x3(h.T.reshape(32, 64, 64), g_w2, g_b2)
    h = _conv3x3(h.T.reshape(64, 64, 64), g_w3, g_b3)
    h = _conv3x3(h.T.reshape(128, 64, 64), g_w4, g_b4)
    gfeat_vec = jnp.mean(h, axis=0)
    gfeat = jnp.broadcast_to(gfeat_vec[None, :], (N_NODES, 16))

    src0 = edge_index[0]
    dst0 = edge_index[1]
    m = jnp.concatenate([mesh_feat[dst0], mesh_feat[src0]], axis=1)
    eh = _matmul(m, l_w1, l_b1, act=True, bm=4000)
    eh = _matmul(eh, l_w2, l_b2, act=True, bm=4000)
    eh = _matmul(eh, l_w3, l_b3, act=True, bm=4000)
    lfeat = jax.ops.segment_sum(eh, dst0, num_segments=N_NODES)

    hidden_in = jnp.concatenate([x[:, 2:], lfeat, gfeat], axis=1)
    hidden = _matmul(hidden_in, lin_w, lin_b, act=True, bm=1000)

    xin = jnp.concatenate([coord, hidden], axis=1)
    xin_p = jnp.pad(xin, ((0, NPAD - N_NODES), (0, 0)))
    xlp = _matmul(xin_p, a_wl, a_bl, bm=1024)
    xrp = _matmul(xin_p, a_wr, a_br, bm=1024)

    loops = jnp.arange(N_NODES, dtype=edge_index.dtype)
    e_src = jnp.concatenate([src0, loops])
    e_dst = jnp.concatenate([dst0, loops])
    order = jnp.argsort(e_dst)
    srcs_s = e_src[order].astype(jnp.int32)
    dsts_s = e_dst[order].astype(jnp.int32)
    srcs_p = jnp.concatenate(
        [srcs_s, jnp.zeros((E_PAD - E_TOT,), jnp.int32)])
    dsts_p = jnp.concatenate(
        [dsts_s, jnp.full((E_PAD - E_TOT,), N_NODES, jnp.int32)])

    gat, pacc, pden = _gat_sparsecore(
        xlp, xrp, a_att.reshape(HH), srcs_p, dsts_p)
    gat = _combine_partials(gat, pacc, pden)

    return _tail(gat, a_bias, c_w, c_b, coord)


# speedup vs baseline: 1.1685x; 1.0089x over previous
import functools

import jax
import jax.numpy as jnp
from jax import lax
from jax.experimental import pallas as pl
from jax.experimental.pallas import tpu as pltpu
from jax.experimental.pallas import tpu_sc as plsc

N_NODES = 10000
N_EDGES = 160000
HEADS = 6
HID = 512
HH = HEADS * HID
NPAD = 10240

NW = 32
E_TOT = N_EDGES + N_NODES
EPT = ((E_TOT + NW * 32 - 1) // (NW * 32)) * 32
E_PAD = EPT * NW
GROUPS = EPT // 16
CCH = HID // 16

_SELU_SCALE = 1.0507009873554805
_SELU_ALPHA = 1.6732632423543772


def _selu(y):
    return _SELU_SCALE * jnp.where(y > 0, y, _SELU_ALPHA * (jnp.exp(y) - 1.0))


def _mm_body(x_ref, w_ref, b_ref, o_ref, *, act):
    y = jnp.dot(x_ref[...], w_ref[...], preferred_element_type=jnp.float32)
    y = y + b_ref[...]
    if act:
        y = _selu(y)
    o_ref[...] = y


def _matmul(x, w, b, act=False, bm=None):
    m, k = x.shape
    n = w.shape[1]
    if bm is None or bm >= m:
        bm = m
    assert m % bm == 0
    return pl.pallas_call(
        functools.partial(_mm_body, act=act),
        grid=(m // bm,),
        in_specs=[
            pl.BlockSpec((bm, k), lambda i: (i, 0)),
            pl.BlockSpec((k, n), lambda i: (0, 0)),
            pl.BlockSpec((1, n), lambda i: (0, 0)),
        ],
        out_specs=pl.BlockSpec((bm, n), lambda i: (i, 0)),
        out_shape=jax.ShapeDtypeStruct((m, n), jnp.float32),
    )(x, w, b.reshape(1, n))


def _conv3x3(img, w, b):
    c = img.shape[0]
    o = w.shape[0]
    xp = jnp.pad(img, ((0, 0), (1, 1), (1, 1)))
    cols = [xp[:, dy:dy + 64, dx:dx + 64] for dy in range(3) for dx in range(3)]
    pat = jnp.stack(cols, axis=1).reshape(c * 9, 4096).T
    wm = w.reshape(o, c * 9).T
    y = _matmul(pat, wm, b, act=True)
    return y


def _tail_body(g_ref, bias_ref, cw_ref, cb_ref, coord_ref, o_ref):
    h = _selu(g_ref[...] / HEADS + bias_ref[...])
    oc = jnp.dot(h, cw_ref[...], preferred_element_type=jnp.float32)
    oc = oc + cb_ref[...]
    coord = coord_ref[...]
    upper = coord[:, 0:1] == 1.0
    down = coord[:, 0:1] == 0.0
    left = coord[:, 1:2] == 0.0
    right = coord[:, 1:2] == 1.0
    ox = jnp.where(upper, 1.0, oc[:, 0:1])
    ox = jnp.where(down, 0.0, ox)
    oy = jnp.where(left, 0.0, oc[:, 1:2])
    oy = jnp.where(right, 1.0, oy)
    o_ref[...] = jnp.concatenate([ox, oy], axis=1)


def _tail(gat, a_bias, c_w, c_b, coord):
    bm = 1000
    return pl.pallas_call(
        _tail_body,
        grid=(N_NODES // bm,),
        in_specs=[
            pl.BlockSpec((bm, HID), lambda i: (i, 0)),
            pl.BlockSpec((1, HID), lambda i: (0, 0)),
            pl.BlockSpec((HID, 2), lambda i: (0, 0)),
            pl.BlockSpec((1, 2), lambda i: (0, 0)),
            pl.BlockSpec((bm, 2), lambda i: (i, 0)),
        ],
        out_specs=pl.BlockSpec((bm, 2), lambda i: (i, 0)),
        out_shape=jax.ShapeDtypeStruct((N_NODES, 2), jnp.float32),
    )(gat, a_bias.reshape(1, HID), c_w, c_b.reshape(1, 2), coord)


def _gat_sc_body(xl_hbm, xr_hbm, att_hbm, src_hbm, dst_hbm,
                 out_hbm, pacc_hbm, pden_hbm,
                 idx_v, dstv_v, u_buf, v_buf, att_v, acc_v, den_v, orow_v,
                 state_s, sem, sem2):
    wid = lax.axis_index("s") * 2 + lax.axis_index("c")
    base = wid * EPT
    zero16 = jnp.zeros((16,), jnp.float32)

    pltpu.sync_copy(src_hbm.at[pl.ds(base, EPT)], idx_v)
    pltpu.sync_copy(dst_hbm.at[pl.ds(base, EPT)], dstv_v.at[pl.ds(0, EPT)])
    pltpu.sync_copy(att_hbm, att_v)

    def _zero_acc():
        def zb(i, _):
            acc_v[pl.ds(i * 16, 16)] = zero16
            return 0
        lax.fori_loop(0, HH // 16, zb, 0)
        for h in range(HEADS):
            den_v[pl.ds(h * 16, 16)] = zero16

    _zero_acc()
    d0 = dstv_v[pl.ds(0, 16)][0]
    state_s[0] = d0
    state_s[1] = 0
    pltpu.sync_copy(xr_hbm.at[d0], v_buf)

    def _write_partial(cur, slot):
        pltpu.sync_copy(acc_v, pacc_hbm.at[slot])
        den_v[pl.ds(96, 16)] = zero16 + cur.astype(jnp.float32)
        den_v[pl.ds(112, 16)] = zero16 + 1.0
        pltpu.sync_copy(den_v, pden_hbm.at[slot])

    def _write_final(cur):
        def cb(c, _):
            s = zero16
            for h in range(HEADS):
                s = s + acc_v[pl.ds(h * HID + c * 16, 16)] / den_v[pl.ds(h * 16, 16)]
            orow_v[pl.ds(c * 16, 16)] = s
            return 0
        lax.fori_loop(0, CCH, cb, 0)
        pltpu.sync_copy(orow_v, out_hbm.at[cur])

    def _make_edge(slot):
        def _edge(j, g):
            eidx = pl.multiple_of(g * 16, 16) + j
            d = dstv_v[pl.ds(eidx, 16)][0]
            cur = state_s[0]

            @pl.when(d != cur)
            def _change():
                first = state_s[1] == 0

                def br_first(_):
                    _write_partial(cur, 2 * wid)
                    return 0

                def br_final(_):
                    _write_final(cur)
                    return 0

                lax.cond(first, br_first, br_final, 0)
                state_s[1] = 1
                _zero_acc()
                pltpu.sync_copy(xr_hbm.at[d], v_buf)
                state_s[0] = d

            for h in range(HEADS):
                hoff = h * HID
                ps = [zero16, zero16, zero16, zero16]
                for c in range(CCH):
                    off = hoff + c * 16
                    uu = u_buf[slot, j, pl.ds(off, 16)]
                    vv = v_buf[pl.ds(off, 16)]
                    aa = att_v[pl.ds(off, 16)]
                    s = uu + vv
                    s = jnp.maximum(s, 0.2 * s)
                    ps[c % 4] = ps[c % 4] + aa * s
                logit = jnp.sum((ps[0] + ps[1]) + (ps[2] + ps[3]))
                w = jnp.exp(zero16 + logit)
                den_v[pl.ds(h * 16, 16)] = den_v[pl.ds(h * 16, 16)] + w
                for c in range(CCH):
                    off = hoff + c * 16
                    acc_v[pl.ds(off, 16)] = acc_v[pl.ds(off, 16)] + w * u_buf[slot, j, pl.ds(off, 16)]
            return g
        return _edge

    _edge0 = _make_edge(0)

    def _group(g, _):
        ivec = idx_v[pl.ds(pl.multiple_of(g * 16, 16), 16)]
        pltpu.async_copy(xl_hbm.at[ivec], u_buf.at[0], sem).wait()
        lax.fori_loop(0, 16, _edge0, g)
        return 0

    lax.fori_loop(0, GROUPS, _group, 0)

    _write_partial(state_s[0], 2 * wid + 1)

    @pl.when(state_s[1] == 0)
    def _no_flush():
        def zb(i, _):
            acc_v[pl.ds(i * 16, 16)] = zero16
            return 0
        lax.fori_loop(0, HH // 16, zb, 0)
        pltpu.sync_copy(acc_v, pacc_hbm.at[2 * wid])
        den_v[pl.ds(96, 16)] = zero16
        den_v[pl.ds(112, 16)] = zero16
        pltpu.sync_copy(den_v, pden_hbm.at[2 * wid])


def _gat_sparsecore(xlp, xrp, att_flat, srcs_p, dsts_p):
    mesh = plsc.VectorSubcoreMesh(core_axis_name="c", subcore_axis_name="s")
    f32 = jnp.float32
    kern = pl.kernel(
        _gat_sc_body,
        out_type=[
            jax.ShapeDtypeStruct((N_NODES, HID), f32),
            jax.ShapeDtypeStruct((2 * NW, HH), f32),
            jax.ShapeDtypeStruct((2 * NW, 128), f32),
        ],
        mesh=mesh,
        compiler_params=pltpu.CompilerParams(needs_layout_passes=False),
        scratch_types=[
            pltpu.VMEM((EPT,), jnp.int32),
            pltpu.VMEM((EPT + 16,), jnp.int32),
            pltpu.VMEM((2, 16, HH), f32),
            pltpu.VMEM((HH,), f32),
            pltpu.VMEM((HH,), f32),
            pltpu.VMEM((HH,), f32),
            pltpu.VMEM((128,), f32),
            pltpu.VMEM((HID,), f32),
            pltpu.SMEM((8,), jnp.int32),
            pltpu.SemaphoreType.DMA,
            pltpu.SemaphoreType.DMA,
        ],
    )
    return kern(xlp, xrp, att_flat, srcs_p, dsts_p)


def _seg_sc_body(msg_hbm, dst_hbm, out_hbm, plf_hbm,
                 dstv_v, msg_v, orow_v, prow_v, state_s):
    wid = lax.axis_index("s") * 2 + lax.axis_index("c")
    base = wid * EPT
    zero16 = jnp.zeros((16,), jnp.float32)

    pltpu.sync_copy(dst_hbm.at[pl.ds(base, EPT)], dstv_v.at[pl.ds(0, EPT)])
    pltpu.sync_copy(msg_hbm.at[pl.ds(base * 16, EPT * 16)], msg_v)

    d0 = dstv_v[pl.ds(0, 16)][0]
    state_s[0] = d0
    state_s[1] = 0

    def _partial(a, cur, slot):
        prow_v[pl.ds(0, 16)] = a
        prow_v[pl.ds(16, 16)] = zero16 + cur.astype(jnp.float32)
        prow_v[pl.ds(32, 16)] = zero16 + 1.0
        pltpu.sync_copy(prow_v, plf_hbm.at[slot])

    def _eb(e, acc):
        d = dstv_v[pl.ds(e, 16)][0]
        cur = state_s[0]

        def chg(a):
            first = state_s[1] == 0

            def bf(_):
                _partial(a, cur, 2 * wid)
                return 0

            def bfin(_):
                orow_v[...] = a
                pltpu.sync_copy(orow_v, out_hbm.at[cur])
                return 0

            lax.cond(first, bf, bfin, 0)
            state_s[1] = 1
            state_s[0] = d
            return zero16

        acc = lax.cond(d != cur, chg, lambda a: a, acc)
        return acc + msg_v[pl.ds(e * 16, 16)]

    accf = lax.fori_loop(0, EPT, _eb, zero16)
    _partial(accf, state_s[0], 2 * wid + 1)

    @pl.when(state_s[1] == 0)
    def _no_flush():
        prow_v[pl.ds(0, 16)] = zero16
        prow_v[pl.ds(16, 16)] = zero16
        prow_v[pl.ds(32, 16)] = zero16
        pltpu.sync_copy(prow_v, plf_hbm.at[2 * wid])


def _seg_sparsecore(msgs_flat, dsts_p):
    mesh = plsc.VectorSubcoreMesh(core_axis_name="c", subcore_axis_name="s")
    f32 = jnp.float32
    kern = pl.kernel(
        _seg_sc_body,
        out_type=[
            jax.ShapeDtypeStruct((N_NODES, 16), f32),
            jax.ShapeDtypeStruct((2 * NW, 48), f32),
        ],
        mesh=mesh,
        compiler_params=pltpu.CompilerParams(needs_layout_passes=False),
        scratch_types=[
            pltpu.VMEM((EPT + 16,), jnp.int32),
            pltpu.VMEM((EPT * 16,), f32),
            pltpu.VMEM((16,), f32),
            pltpu.VMEM((48,), f32),
            pltpu.SMEM((8,), jnp.int32),
        ],
    )
    return kern(msgs_flat, dsts_p)


def _combine_seg(lf, plf):
    acc = plf[:, 0:16]
    pdst = plf[:, 16].astype(jnp.int32)
    okay = (plf[:, 32] > 0.5) & (pdst >= 0) & (pdst < N_NODES)
    eq = (pdst[:, None] == pdst[None, :]) & okay[:, None] & okay[None, :]
    rows = eq.astype(jnp.float32) @ acc
    safe = jnp.where(okay, pdst, N_NODES)
    return lf.at[safe].set(rows, mode="drop")


def _combine_partials(gat, pacc, pden):
    den = pden[:, 0:HEADS * 16:16]
    pdst = pden[:, 96].astype(jnp.int32)
    valid = pden[:, 112] > 0.5
    okay = valid & (pdst >= 0) & (pdst < N_NODES)
    eq = (pdst[:, None] == pdst[None, :]) & okay[:, None] & okay[None, :]
    eqf = eq.astype(jnp.float32)
    acc_c = eqf @ pacc
    den_c = eqf @ den
    den_c = jnp.where(den_c == 0.0, 1.0, den_c)
    rows = (acc_c.reshape(-1, HEADS, HID) / den_c[:, :, None]).sum(axis=1)
    safe = jnp.where(okay, pdst, N_NODES)
    return gat.at[safe].set(rows, mode="drop")


def kernel(x, conv_feat, mesh_feat, edge_index, node_num, bd_mask, poly_mesh,
           g_w1, g_b1, g_w2, g_b2, g_w3, g_b3, g_w4, g_b4,
           l_w1, l_b1, l_w2, l_b2, l_w3, l_b3, lin_w, lin_b,
           a_wl, a_bl, a_wr, a_br, a_att, a_bias, c_w, c_b):
    coord = x[:, :2]

    h = _conv3x3(conv_feat[0], g_w1, g_b1)
    h = _conv3x3(h.T.reshape(32, 64, 64), g_w2, g_b2)
    h = _conv3x3(h.T.reshape(64, 64, 64), g_w3, g_b3)
    h = _conv3x3(h.T.reshape(128, 64, 64), g_w4, g_b4)
    gfeat_vec = jnp.mean(h, axis=0)
    gfeat = jnp.broadcast_to(gfeat_vec[None, :], (N_NODES, 16))

    src0 = edge_index[0]
    dst0 = edge_index[1]
    loops = jnp.arange(N_NODES, dtype=edge_index.dtype)
    e_src = jnp.concatenate([src0, loops])
    e_dst = jnp.concatenate([dst0, loops])
    order = jnp.argsort(e_dst)
    srcs_s = e_src[order].astype(jnp.int32)
    dsts_s = e_dst[order].astype(jnp.int32)
    srcs_p = jnp.concatenate(
        [srcs_s, jnp.zeros((E_PAD - E_TOT,), jnp.int32)])
    dsts_p = jnp.concatenate(
        [dsts_s, jnp.full((E_PAD - E_TOT,), N_NODES, jnp.int32)])
    real_edge = jnp.concatenate(
        [order < N_EDGES, jnp.zeros((E_PAD - E_TOT,), bool)])

    m = jnp.concatenate([mesh_feat[dsts_s[:, None].clip(0, N_NODES - 1)][:, 0],
                         mesh_feat[srcs_s[:, None].clip(0, N_NODES - 1)][:, 0]],
                        axis=1)
    m = jnp.pad(m, ((0, E_PAD - E_TOT), (0, 0)))
    eh = _matmul(m, l_w1, l_b1, act=True, bm=5344)
    eh = _matmul(eh, l_w2, l_b2, act=True, bm=5344)
    eh = _matmul(eh, l_w3, l_b3, act=True, bm=5344)
    eh = eh * real_edge[:, None].astype(jnp.float32)
    lf, plf = _seg_sparsecore(eh.reshape(-1), dsts_p)
    lfeat = _combine_seg(lf, plf)

    hidden_in = jnp.concatenate([x[:, 2:], lfeat, gfeat], axis=1)
    hidden = _matmul(hidden_in, lin_w, lin_b, act=True, bm=1000)

    xin = jnp.concatenate([coord, hidden], axis=1)
    xin_p = jnp.pad(xin, ((0, NPAD - N_NODES), (0, 0)))
    xlp = _matmul(xin_p, a_wl, a_bl, bm=1024)
    xrp = _matmul(xin_p, a_wr, a_br, bm=1024)

    gat, pacc, pden = _gat_sparsecore(
        xlp, xrp, a_att.reshape(HH), srcs_p, dsts_p)
    gat = _combine_partials(gat, pacc, pden)

    return _tail(gat, a_bias, c_w, c_b, coord)
